# baseline (speedup 1.0000x reference)
import functools

import jax
import jax.numpy as jnp
from jax import lax
from jax.experimental import pallas as pl
from jax.experimental.pallas import tpu as pltpu
from jax.experimental.pallas import tpu_sc as plsc

N = 10000
D = 128
DH = D // 2
NPAD = 10240
BLK = 1000
STRIPE = NPAD // 16
K = 128
H2 = 100000
H3 = 40000
NCH2 = 50
NCH3 = 20
HT2 = NCH2 * K
HT3 = NCH3 * K
E2P = 16 * HT2 * 2
E3P = 16 * HT3 * 3


def _z_body(x_ref, a2_ref, a3_ref, *z_refs):
    xb = x_ref[...]
    zb = [lax.dot_general(xb, a2_ref[t * D:(t + 1) * D, :],
                          (((1,), (0,)), ((), ())),
                          preferred_element_type=jnp.float32)
          for t in range(2)]
    zb += [lax.dot_general(xb, a3_ref[t * D:(t + 1) * D, :],
                           (((1,), (0,)), ((), ())),
                           preferred_element_type=jnp.float32)
           for t in range(3)]
    for i in range(5):
        for h in range(2):
            z_refs[i][h] = zb[i][:, h * DH:(h + 1) * DH]


_z_kernel = pl.pallas_call(
    _z_body,
    grid=(N // BLK,),
    in_specs=[
        pl.BlockSpec((BLK, D), lambda i: (i, 0)),
        pl.BlockSpec((2 * D, D), lambda i: (0, 0)),
        pl.BlockSpec((3 * D, D), lambda i: (0, 0)),
    ],
    out_specs=[pl.BlockSpec((2, BLK, DH), lambda i: (0, i, 0))] * 5,
    out_shape=[jax.ShapeDtypeStruct((2, N, DH), jnp.float32)] * 5,
)


@functools.lru_cache(maxsize=None)
def _get_sc_scatter():
  mesh = plsc.VectorSubcoreMesh(core_axis_name="c", subcore_axis_name="s")

  @functools.partial(
    pl.kernel,
    out_type=(
        jax.ShapeDtypeStruct((2, NPAD, D), jnp.float32),
        jax.ShapeDtypeStruct((2, NPAD, 16), jnp.float32),
    ),
    mesh=mesh,
    compiler_params=pltpu.CompilerParams(use_tc_tiling_on_sc=False, needs_layout_passes=False),
    scratch_types=(
        pltpu.VMEM((2 * HT2,), jnp.int32),
        pltpu.VMEM((2 * HT2,), jnp.int32),
        pltpu.VMEM((2 * NCH2, K), jnp.int32),
        pltpu.VMEM((NCH2, K), jnp.int32),
        pltpu.VMEM((32,), jnp.int32),
        pltpu.VMEM((K, DH), jnp.float32),
        pltpu.VMEM((K, DH), jnp.float32),
        pltpu.VMEM((K, 16), jnp.float32),
        pltpu.VMEM((K, 16), jnp.float32),
        pltpu.VMEM_SHARED((NPAD, DH), jnp.float32),
        pltpu.VMEM_SHARED((NPAD, 16), jnp.float32),
        pltpu.SemaphoreType.DMA,
        pltpu.SemaphoreType.DMA,
        pltpu.SemaphoreType.DMA,
    ),
  )
  def _sc_scatter(z2s0, z2s1, z3s0, z3s1, z3s2, e2_hbm, e3_hbm,
                  w_out, cnt_out,
                  e0_v, e1_v, gsrc_v, gdst_v, stg_v, rows0_v, rows1_v,
                  zero16_v, ones_v, w_sh, cnt_sh, sem0, sem1, semc):
      c = lax.axis_index("c")
      s = lax.axis_index("s")
      row0 = s * STRIPE
      iota = lax.iota(jnp.int32, 16)

      def _fill(ref, val):
          def body(i, carry):
              for k in range(ref.shape[1] // 16):
                  ref[i, pl.ds(k * 16, 16)] = jnp.full((16,), val, jnp.float32)
              return carry
          lax.fori_loop(0, ref.shape[0], body, 0)

      _fill(zero16_v, 0.0)
      _fill(ones_v, 1.0)

      def _zero_acc():
          _fill(rows0_v, 0.0)
          for p in range(STRIPE // K):
              pltpu.sync_copy(rows0_v, w_sh.at[pl.ds(row0 + p * K, K)])
              pltpu.sync_copy(zero16_v, cnt_sh.at[pl.ds(row0 + p * K, K)])

      _zero_acc()
      plsc.subcore_barrier()

      for rel, (e_hbm, ht, nch, arity, hreal, ztabs) in enumerate((
              (e2_hbm, HT2, NCH2, 2, H2, (z2s0, z2s1)),
              (e3_hbm, HT3, NCH3, 3, H3, (z3s0, z3s1, z3s2)))):
          span = arity * ht
          with jax.named_scope(f"spanload{rel}"):
              pltpu.sync_copy(e_hbm.at[0, pl.ds(s * span, span)],
                              e0_v.at[pl.ds(0, span)])
              pltpu.sync_copy(e_hbm.at[1, pl.ds(s * span, span)],
                              e1_v.at[pl.ds(0, span)])
          masks = [(iota + 16 * u) % arity == t
                   for u in range(arity) for t in range(arity)]
          offs = [0] * (arity * arity)
          for t in range(arity):
              acc = 0
              for u in range(arity):
                  offs[u * arity + t] = acc
                  acc += sum(1 for l in range(16) if (16 * u + l) % arity == t)

          def _build(w, carry):
              base = w * 16 * arity
              row = w // 8
              col = 16 * (w % 8)
              vs = [e0_v[pl.ds(base + 16 * u, 16)] for u in range(arity)]
              ws = [e1_v[pl.ds(base + 16 * u, 16)] for u in range(arity)]
              for t in range(arity):
                  for u in range(arity):
                      plsc.store_compressed(
                          stg_v.at[pl.ds(offs[u * arity + t], 16)],
                          jnp.minimum(vs[u], N - 1),
                          mask=masks[u * arity + t])
                  gsrc_v[t * nch + row, pl.ds(col, 16)] = stg_v[pl.ds(0, 16)]
              for u in range(arity):
                  plsc.store_compressed(stg_v.at[pl.ds(offs[u * arity], 16)],
                                        ws[u], mask=masks[u * arity])
              dv = stg_v[pl.ds(0, 16)]
              dv = jnp.where(dv == N, N + col + iota, dv)
              gdst_v[row, pl.ds(col, 16)] = dv
              return carry

          with jax.named_scope(f"build{rel}"):
              lax.fori_loop(0, ht // 16, _build, 0)

          for t in range(arity):
              zt = ztabs[t].at[c]
              toff = t * nch
              pltpu.async_copy(zt.at[gsrc_v.at[toff]], rows0_v, sem0)

              def _pair(g, carry):
                  j0 = 2 * g
                  if t == 0:
                      @pl.when(c == rel)
                      def _():
                          pltpu.async_copy(ones_v, cnt_sh.at[gdst_v.at[j0]],
                                           semc, add=True)
                          pltpu.async_copy(ones_v,
                                           cnt_sh.at[gdst_v.at[j0 + 1]],
                                           semc, add=True)

                  cp1 = pltpu.async_copy(zt.at[gsrc_v.at[toff + j0 + 1]],
                                         rows1_v, sem1)
                  pltpu.make_async_copy(zt.at[pl.ds(0, K)], rows0_v,
                                        sem0).wait()
                  pltpu.sync_copy(rows0_v, w_sh.at[gdst_v.at[j0]], add=True)

                  @pl.when(j0 + 2 < nch)
                  def _():
                      pltpu.async_copy(zt.at[gsrc_v.at[toff + j0 + 2]],
                                       rows0_v, sem0)

                  cp1.wait()
                  pltpu.sync_copy(rows1_v, w_sh.at[gdst_v.at[j0 + 1]],
                                  add=True)

                  if t == 0:
                      @pl.when(c == rel)
                      def _():
                          pltpu.make_async_copy(ones_v,
                                                cnt_sh.at[pl.ds(0, K)],
                                                semc).wait()
                          pltpu.make_async_copy(ones_v,
                                                cnt_sh.at[pl.ds(0, K)],
                                                semc).wait()
                  return carry

              with jax.named_scope(f"stream{rel}_{t}"):
                  lax.fori_loop(0, nch // 2, _pair, 0)

          plsc.subcore_barrier()
          pltpu.sync_copy(
              w_sh.at[pl.ds(row0, STRIPE)],
              w_out.at[rel, pl.ds(row0, STRIPE), pl.ds(c * DH, DH)])

          @pl.when(c == rel)
          def _():
              pltpu.sync_copy(cnt_sh.at[pl.ds(row0, STRIPE)],
                              cnt_out.at[rel, pl.ds(row0, STRIPE)])

          if rel == 0:
              _zero_acc()
              plsc.subcore_barrier()

  return _sc_scatter


def _final_body(x_ref, w_ref, cnt_ref, cw_ref, cb_ref, o_ref):
    xb = x_ref[...]
    c2 = cnt_ref[0][:, 0]
    c3 = cnt_ref[1][:, 0]
    r2 = jnp.where(c2 > 0, 1.0 / c2, 0.0)
    r3 = jnp.where(c3 > 0, 1.0 / c3, 0.0)
    agg = r2[:, None] * w_ref[0] + r3[:, None] * w_ref[1]
    h = lax.dot_general(xb, cw_ref[...], (((1,), (1,)), ((), ())),
                        preferred_element_type=jnp.float32)
    o_ref[...] = h + cb_ref[...] + agg


_final_kernel = pl.pallas_call(
    _final_body,
    grid=(N // BLK,),
    in_specs=[
        pl.BlockSpec((BLK, D), lambda i: (i, 0)),
        pl.BlockSpec((2, BLK, D), lambda i: (0, i, 0)),
        pl.BlockSpec((2, BLK, 16), lambda i: (0, i, 0)),
        pl.BlockSpec((D, D), lambda i: (0, 0)),
        pl.BlockSpec((1, D), lambda i: (0, 0)),
    ],
    out_specs=pl.BlockSpec((BLK, D), lambda i: (i, 0)),
    out_shape=jax.ShapeDtypeStruct((N, D), jnp.float32),
)


def kernel(x, edge_index_rel2, edge_index_rel3, A_rel2, A_rel3, C_w, C_b):
    z = _z_kernel(x, A_rel2, A_rel3)
    e2p = jnp.pad(edge_index_rel2, ((0, 0), (0, E2P - 2 * H2)),
                  constant_values=N)
    e3p = jnp.pad(edge_index_rel3, ((0, 0), (0, E3P - 3 * H3)),
                  constant_values=N)
    w, cnt = _get_sc_scatter()(*z, e2p, e3p)
    return _final_kernel(x, w, cnt, C_w, C_b.reshape(1, D))


# speedup vs baseline: 1.2627x; 1.2627x over previous
import functools

import jax
import jax.numpy as jnp
from jax import lax
from jax.experimental import pallas as pl
from jax.experimental.pallas import tpu as pltpu
from jax.experimental.pallas import tpu_sc as plsc

N = 10000
D = 128
DH = D // 2
NPAD = 10240
BLK = 1000
STRIPE = NPAD // 16
K = 128
H2 = 100000
H3 = 40000
NCH2 = 50
NCH3 = 20
HT2 = NCH2 * K
HT3 = NCH3 * K
E2P = 16 * HT2 * 2
E3P = 16 * HT3 * 3


def _z_body(x_ref, a2_ref, a3_ref, *z_refs):
    xb = x_ref[...]
    zb = [lax.dot_general(xb, a2_ref[t * D:(t + 1) * D, :],
                          (((1,), (0,)), ((), ())),
                          preferred_element_type=jnp.float32)
          for t in range(2)]
    zb += [lax.dot_general(xb, a3_ref[t * D:(t + 1) * D, :],
                           (((1,), (0,)), ((), ())),
                           preferred_element_type=jnp.float32)
           for t in range(3)]
    for i in range(5):
        zb16 = zb[i].astype(jnp.bfloat16)
        for h in range(2):
            z_refs[i][h] = zb16[:, h * DH:(h + 1) * DH]


_z_kernel = pl.pallas_call(
    _z_body,
    grid=(N // BLK,),
    in_specs=[
        pl.BlockSpec((BLK, D), lambda i: (i, 0)),
        pl.BlockSpec((2 * D, D), lambda i: (0, 0)),
        pl.BlockSpec((3 * D, D), lambda i: (0, 0)),
    ],
    out_specs=[pl.BlockSpec((2, BLK, DH), lambda i: (0, i, 0))] * 5,
    out_shape=[jax.ShapeDtypeStruct((2, N, DH), jnp.bfloat16)] * 5,
)


@functools.lru_cache(maxsize=None)
def _get_sc_scatter():
  mesh = plsc.VectorSubcoreMesh(core_axis_name="c", subcore_axis_name="s")

  @functools.partial(
    pl.kernel,
    out_type=(
        jax.ShapeDtypeStruct((2, NPAD, D), jnp.bfloat16),
        jax.ShapeDtypeStruct((2, NPAD, 16), jnp.float32),
    ),
    mesh=mesh,
    compiler_params=pltpu.CompilerParams(use_tc_tiling_on_sc=False, needs_layout_passes=False),
    scratch_types=(
        pltpu.VMEM((2 * HT2,), jnp.int32),
        pltpu.VMEM((2 * HT2,), jnp.int32),
        pltpu.VMEM((2 * NCH2, K), jnp.int32),
        pltpu.VMEM((NCH2, K), jnp.int32),
        pltpu.VMEM((32,), jnp.int32),
        pltpu.VMEM((K, DH), jnp.bfloat16),
        pltpu.VMEM((K, DH), jnp.bfloat16),
        pltpu.VMEM((K, 16), jnp.float32),
        pltpu.VMEM((K, 16), jnp.float32),
        pltpu.VMEM_SHARED((NPAD, DH), jnp.bfloat16),
        pltpu.VMEM_SHARED((NPAD, 16), jnp.float32),
        pltpu.SemaphoreType.DMA,
        pltpu.SemaphoreType.DMA,
        pltpu.SemaphoreType.DMA,
    ),
  )
  def _sc_scatter(z2s0, z2s1, z3s0, z3s1, z3s2, e2_hbm, e3_hbm,
                  w_out, cnt_out,
                  e0_v, e1_v, gsrc_v, gdst_v, stg_v, rows0_v, rows1_v,
                  zero16_v, ones_v, w_sh, cnt_sh, sem0, sem1, semc):
      c = lax.axis_index("c")
      s = lax.axis_index("s")
      row0 = s * STRIPE
      iota = lax.iota(jnp.int32, 16)

      def _fill(ref, val):
          lanes = 32 if ref.dtype == jnp.bfloat16 else 16
          def body(i, carry):
              for k in range(ref.shape[1] // lanes):
                  ref[i, pl.ds(k * lanes, lanes)] = jnp.full(
                      (lanes,), val, ref.dtype)
              return carry
          lax.fori_loop(0, ref.shape[0], body, 0)

      _fill(zero16_v, 0.0)
      _fill(ones_v, 1.0)

      def _zero_acc():
          _fill(rows0_v, 0.0)
          for p in range(STRIPE // K):
              pltpu.sync_copy(rows0_v, w_sh.at[pl.ds(row0 + p * K, K)])
              pltpu.sync_copy(zero16_v, cnt_sh.at[pl.ds(row0 + p * K, K)])

      _zero_acc()
      plsc.subcore_barrier()

      for rel, (e_hbm, ht, nch, arity, hreal, ztabs) in enumerate((
              (e2_hbm, HT2, NCH2, 2, H2, (z2s0, z2s1)),
              (e3_hbm, HT3, NCH3, 3, H3, (z3s0, z3s1, z3s2)))):
          span = arity * ht
          with jax.named_scope(f"spanload{rel}"):
              pltpu.sync_copy(e_hbm.at[0, pl.ds(s * span, span)],
                              e0_v.at[pl.ds(0, span)])
              pltpu.sync_copy(e_hbm.at[1, pl.ds(s * span, span)],
                              e1_v.at[pl.ds(0, span)])
          masks = [(iota + 16 * u) % arity == t
                   for u in range(arity) for t in range(arity)]
          offs = [0] * (arity * arity)
          for t in range(arity):
              acc = 0
              for u in range(arity):
                  offs[u * arity + t] = acc
                  acc += sum(1 for l in range(16) if (16 * u + l) % arity == t)

          def _build(w, carry):
              base = w * 16 * arity
              row = w // 8
              col = 16 * (w % 8)
              vs = [e0_v[pl.ds(base + 16 * u, 16)] for u in range(arity)]
              ws = [e1_v[pl.ds(base + 16 * u, 16)] for u in range(arity)]
              for t in range(arity):
                  for u in range(arity):
                      plsc.store_compressed(
                          stg_v.at[pl.ds(offs[u * arity + t], 16)],
                          jnp.minimum(vs[u], N - 1),
                          mask=masks[u * arity + t])
                  gsrc_v[t * nch + row, pl.ds(col, 16)] = stg_v[pl.ds(0, 16)]
              for u in range(arity):
                  plsc.store_compressed(stg_v.at[pl.ds(offs[u * arity], 16)],
                                        ws[u], mask=masks[u * arity])
              dv = stg_v[pl.ds(0, 16)]
              dv = jnp.where(dv == N, N + col + iota, dv)
              gdst_v[row, pl.ds(col, 16)] = dv
              return carry

          with jax.named_scope(f"build{rel}"):
              lax.fori_loop(0, ht // 16, _build, 0)

          for t in range(arity):
              zt = ztabs[t].at[c]
              toff = t * nch
              pltpu.async_copy(zt.at[gsrc_v.at[toff]], rows0_v, sem0)

              def _pair(g, carry):
                  j0 = 2 * g
                  if t == 0:
                      @pl.when(c == rel)
                      def _():
                          pltpu.async_copy(ones_v, cnt_sh.at[gdst_v.at[j0]],
                                           semc, add=True)
                          pltpu.async_copy(ones_v,
                                           cnt_sh.at[gdst_v.at[j0 + 1]],
                                           semc, add=True)

                  cp1 = pltpu.async_copy(zt.at[gsrc_v.at[toff + j0 + 1]],
                                         rows1_v, sem1)
                  pltpu.make_async_copy(zt.at[pl.ds(0, K)], rows0_v,
                                        sem0).wait()
                  pltpu.sync_copy(rows0_v, w_sh.at[gdst_v.at[j0]], add=True)

                  @pl.when(j0 + 2 < nch)
                  def _():
                      pltpu.async_copy(zt.at[gsrc_v.at[toff + j0 + 2]],
                                       rows0_v, sem0)

                  cp1.wait()
                  pltpu.sync_copy(rows1_v, w_sh.at[gdst_v.at[j0 + 1]],
                                  add=True)

                  if t == 0:
                      @pl.when(c == rel)
                      def _():
                          pltpu.make_async_copy(ones_v,
                                                cnt_sh.at[pl.ds(0, K)],
                                                semc).wait()
                          pltpu.make_async_copy(ones_v,
                                                cnt_sh.at[pl.ds(0, K)],
                                                semc).wait()
                  return carry

              with jax.named_scope(f"stream{rel}_{t}"):
                  lax.fori_loop(0, nch // 2, _pair, 0)

          plsc.subcore_barrier()
          pltpu.sync_copy(
              w_sh.at[pl.ds(row0, STRIPE)],
              w_out.at[rel, pl.ds(row0, STRIPE), pl.ds(c * DH, DH)])

          @pl.when(c == rel)
          def _():
              pltpu.sync_copy(cnt_sh.at[pl.ds(row0, STRIPE)],
                              cnt_out.at[rel, pl.ds(row0, STRIPE)])

          if rel == 0:
              _zero_acc()
              plsc.subcore_barrier()

  return _sc_scatter


def _final_body(x_ref, w_ref, cnt_ref, cw_ref, cb_ref, o_ref):
    xb = x_ref[...]
    c2 = cnt_ref[0][:, 0]
    c3 = cnt_ref[1][:, 0]
    r2 = jnp.where(c2 > 0, 1.0 / c2, 0.0)
    r3 = jnp.where(c3 > 0, 1.0 / c3, 0.0)
    wf = w_ref[...].astype(jnp.float32)
    agg = r2[:, None] * wf[0] + r3[:, None] * wf[1]
    h = lax.dot_general(xb, cw_ref[...], (((1,), (1,)), ((), ())),
                        preferred_element_type=jnp.float32)
    o_ref[...] = h + cb_ref[...] + agg


_final_kernel = pl.pallas_call(
    _final_body,
    grid=(N // BLK,),
    in_specs=[
        pl.BlockSpec((BLK, D), lambda i: (i, 0)),
        pl.BlockSpec((2, BLK, D), lambda i: (0, i, 0)),
        pl.BlockSpec((2, BLK, 16), lambda i: (0, i, 0)),
        pl.BlockSpec((D, D), lambda i: (0, 0)),
        pl.BlockSpec((1, D), lambda i: (0, 0)),
    ],
    out_specs=pl.BlockSpec((BLK, D), lambda i: (i, 0)),
    out_shape=jax.ShapeDtypeStruct((N, D), jnp.float32),
)


def kernel(x, edge_index_rel2, edge_index_rel3, A_rel2, A_rel3, C_w, C_b):
    z = _z_kernel(x, A_rel2, A_rel3)
    e2p = jnp.pad(edge_index_rel2, ((0, 0), (0, E2P - 2 * H2)),
                  constant_values=N)
    e3p = jnp.pad(edge_index_rel3, ((0, 0), (0, E3P - 3 * H3)),
                  constant_values=N)
    w, cnt = _get_sc_scatter()(*z, e2p, e3p)
    return _final_kernel(x, w, cnt, C_w, C_b.reshape(1, D))


# speedup vs baseline: 1.2748x; 1.0095x over previous
import functools

import jax
import jax.numpy as jnp
from jax import lax
from jax.experimental import pallas as pl
from jax.experimental.pallas import tpu as pltpu
from jax.experimental.pallas import tpu_sc as plsc

N = 10000
D = 128
DH = D // 2
NPAD = 10240
BLK = 1000
STRIPE = NPAD // 16
K = 128
H2 = 100000
H3 = 40000
NCH2 = 50
NCH3 = 20
HT2 = NCH2 * K
HT3 = NCH3 * K
E2P = 16 * HT2 * 2
E3P = 16 * HT3 * 3


def _z_body(x_ref, a2_ref, a3_ref, *z_refs):
    xb = x_ref[...]
    zb = [lax.dot_general(xb, a2_ref[t * D:(t + 1) * D, :],
                          (((1,), (0,)), ((), ())),
                          preferred_element_type=jnp.float32)
          for t in range(2)]
    zb += [lax.dot_general(xb, a3_ref[t * D:(t + 1) * D, :],
                           (((1,), (0,)), ((), ())),
                           preferred_element_type=jnp.float32)
           for t in range(3)]
    z_ref, = z_refs
    for i in range(5):
        zb16 = zb[i].astype(jnp.bfloat16)
        for h in range(2):
            z_ref[i, h] = zb16[:, h * DH:(h + 1) * DH]


_z_kernel = pl.pallas_call(
    _z_body,
    grid=(N // BLK,),
    in_specs=[
        pl.BlockSpec((BLK, D), lambda i: (i, 0)),
        pl.BlockSpec((2 * D, D), lambda i: (0, 0)),
        pl.BlockSpec((3 * D, D), lambda i: (0, 0)),
    ],
    out_specs=[pl.BlockSpec((5, 2, BLK, DH), lambda i: (0, 0, i, 0))],
    out_shape=[jax.ShapeDtypeStruct((5, 2, N, DH), jnp.bfloat16)],
)


@functools.lru_cache(maxsize=None)
def _get_sc_scatter():
  mesh = plsc.VectorSubcoreMesh(core_axis_name="c", subcore_axis_name="s")

  @functools.partial(
    pl.kernel,
    out_type=(
        jax.ShapeDtypeStruct((2, NPAD, D), jnp.bfloat16),
        jax.ShapeDtypeStruct((2, NPAD, 16), jnp.float32),
    ),
    mesh=mesh,
    compiler_params=pltpu.CompilerParams(use_tc_tiling_on_sc=False, needs_layout_passes=False),
    scratch_types=(
        pltpu.VMEM((2 * HT2,), jnp.int32),
        pltpu.VMEM((2 * HT2,), jnp.int32),
        pltpu.VMEM((2 * NCH2, K), jnp.int32),
        pltpu.VMEM((NCH2, K), jnp.int32),
        pltpu.VMEM((32,), jnp.int32),
        pltpu.VMEM((K, DH), jnp.bfloat16),
        pltpu.VMEM((K, DH), jnp.bfloat16),
        pltpu.VMEM((K, 16), jnp.float32),
        pltpu.VMEM((K, 16), jnp.float32),
        pltpu.VMEM_SHARED((NPAD, DH), jnp.bfloat16),
        pltpu.VMEM_SHARED((NPAD, 16), jnp.float32),
        pltpu.SemaphoreType.DMA,
        pltpu.SemaphoreType.DMA,
        pltpu.SemaphoreType.DMA,
    ),
  )
  def _sc_scatter(ztab_hbm, e2_hbm, e3_hbm,
                  w_out, cnt_out,
                  e0_v, e1_v, gsrc_v, gdst_v, stg_v, rows0_v, rows1_v,
                  zero16_v, ones_v, w_sh, cnt_sh, sem0, sem1, semc):
      c = lax.axis_index("c")
      s = lax.axis_index("s")
      row0 = s * STRIPE
      iota = lax.iota(jnp.int32, 16)

      def _fill(ref, val):
          lanes = 32 if ref.dtype == jnp.bfloat16 else 16
          def body(i, carry):
              for k in range(ref.shape[1] // lanes):
                  ref[i, pl.ds(k * lanes, lanes)] = jnp.full(
                      (lanes,), val, ref.dtype)
              return carry
          lax.fori_loop(0, ref.shape[0], body, 0)

      _fill(zero16_v, 0.0)
      _fill(ones_v, 1.0)

      def _zero_acc():
          _fill(rows0_v, 0.0)
          for p in range(STRIPE // K):
              pltpu.sync_copy(rows0_v, w_sh.at[pl.ds(row0 + p * K, K)])
              pltpu.sync_copy(zero16_v, cnt_sh.at[pl.ds(row0 + p * K, K)])

      _zero_acc()
      plsc.subcore_barrier()

      for rel, (e_hbm, ht, nch, arity, hreal, tab0) in enumerate((
              (e2_hbm, HT2, NCH2, 2, H2, 0),
              (e3_hbm, HT3, NCH3, 3, H3, 2))):
          span = arity * ht
          with jax.named_scope(f"spanload{rel}"):
              pltpu.sync_copy(e_hbm.at[0, pl.ds(s * span, span)],
                              e0_v.at[pl.ds(0, span)])
              pltpu.sync_copy(e_hbm.at[1, pl.ds(s * span, span)],
                              e1_v.at[pl.ds(0, span)])
          masks = [(iota + 16 * u) % arity == t
                   for u in range(arity) for t in range(arity)]
          offs = [0] * (arity * arity)
          for t in range(arity):
              acc = 0
              for u in range(arity):
                  offs[u * arity + t] = acc
                  acc += sum(1 for l in range(16) if (16 * u + l) % arity == t)

          def _build(w, carry):
              base = w * 16 * arity
              row = w // 8
              col = 16 * (w % 8)
              vs = [e0_v[pl.ds(base + 16 * u, 16)] for u in range(arity)]
              ws = [e1_v[pl.ds(base + 16 * u, 16)] for u in range(arity)]
              for t in range(arity):
                  for u in range(arity):
                      plsc.store_compressed(
                          stg_v.at[pl.ds(offs[u * arity + t], 16)],
                          jnp.minimum(vs[u], N - 1),
                          mask=masks[u * arity + t])
                  gsrc_v[t * nch + row, pl.ds(col, 16)] = stg_v[pl.ds(0, 16)]
              for u in range(arity):
                  plsc.store_compressed(stg_v.at[pl.ds(offs[u * arity], 16)],
                                        ws[u], mask=masks[u * arity])
              dv = stg_v[pl.ds(0, 16)]
              dv = jnp.where(dv == N, N + col + iota, dv)
              gdst_v[row, pl.ds(col, 16)] = dv
              return carry

          with jax.named_scope(f"build{rel}"):
              lax.fori_loop(0, ht // 16, _build, 0)

          for t in range(arity):
              zt = ztab_hbm.at[tab0 + t].at[c]
              toff = t * nch
              pltpu.async_copy(zt.at[gsrc_v.at[toff]], rows0_v, sem0)

              def _pair(g, carry):
                  j0 = 2 * g
                  if t == 0:
                      @pl.when(c == rel)
                      def _():
                          pltpu.async_copy(ones_v, cnt_sh.at[gdst_v.at[j0]],
                                           semc, add=True)
                          pltpu.async_copy(ones_v,
                                           cnt_sh.at[gdst_v.at[j0 + 1]],
                                           semc, add=True)

                  cp1 = pltpu.async_copy(zt.at[gsrc_v.at[toff + j0 + 1]],
                                         rows1_v, sem1)
                  pltpu.make_async_copy(zt.at[pl.ds(0, K)], rows0_v,
                                        sem0).wait()
                  pltpu.sync_copy(rows0_v, w_sh.at[gdst_v.at[j0]], add=True)

                  @pl.when(j0 + 2 < nch)
                  def _():
                      pltpu.async_copy(zt.at[gsrc_v.at[toff + j0 + 2]],
                                       rows0_v, sem0)

                  cp1.wait()
                  pltpu.sync_copy(rows1_v, w_sh.at[gdst_v.at[j0 + 1]],
                                  add=True)

                  if t == 0:
                      @pl.when(c == rel)
                      def _():
                          pltpu.make_async_copy(ones_v,
                                                cnt_sh.at[pl.ds(0, K)],
                                                semc).wait()
                          pltpu.make_async_copy(ones_v,
                                                cnt_sh.at[pl.ds(0, K)],
                                                semc).wait()
                  return carry

              with jax.named_scope(f"stream{rel}_{t}"):
                  lax.fori_loop(0, nch // 2, _pair, 0)

          plsc.subcore_barrier()
          pltpu.sync_copy(
              w_sh.at[pl.ds(row0, STRIPE)],
              w_out.at[rel, pl.ds(row0, STRIPE), pl.ds(c * DH, DH)])

          @pl.when(c == rel)
          def _():
              pltpu.sync_copy(cnt_sh.at[pl.ds(row0, STRIPE)],
                              cnt_out.at[rel, pl.ds(row0, STRIPE)])

          if rel == 0:
              _zero_acc()
              plsc.subcore_barrier()

  return _sc_scatter


def _final_body(x_ref, w_ref, cnt_ref, cw_ref, cb_ref, o_ref):
    xb = x_ref[...]
    c2 = cnt_ref[0][:, 0]
    c3 = cnt_ref[1][:, 0]
    r2 = jnp.where(c2 > 0, 1.0 / c2, 0.0)
    r3 = jnp.where(c3 > 0, 1.0 / c3, 0.0)
    wf = w_ref[...].astype(jnp.float32)
    agg = r2[:, None] * wf[0] + r3[:, None] * wf[1]
    h = lax.dot_general(xb, cw_ref[...], (((1,), (1,)), ((), ())),
                        preferred_element_type=jnp.float32)
    o_ref[...] = h + cb_ref[...] + agg


_final_kernel = pl.pallas_call(
    _final_body,
    grid=(N // BLK,),
    in_specs=[
        pl.BlockSpec((BLK, D), lambda i: (i, 0)),
        pl.BlockSpec((2, BLK, D), lambda i: (0, i, 0)),
        pl.BlockSpec((2, BLK, 16), lambda i: (0, i, 0)),
        pl.BlockSpec((D, D), lambda i: (0, 0)),
        pl.BlockSpec((1, D), lambda i: (0, 0)),
    ],
    out_specs=pl.BlockSpec((BLK, D), lambda i: (i, 0)),
    out_shape=jax.ShapeDtypeStruct((N, D), jnp.float32),
)


def kernel(x, edge_index_rel2, edge_index_rel3, A_rel2, A_rel3, C_w, C_b):
    z, = _z_kernel(x, A_rel2, A_rel3)
    e2p = jnp.pad(edge_index_rel2, ((0, 0), (0, E2P - 2 * H2)),
                  constant_values=N)
    e3p = jnp.pad(edge_index_rel3, ((0, 0), (0, E3P - 3 * H3)),
                  constant_values=N)
    w, cnt = _get_sc_scatter()(z, e2p, e3p)
    return _final_kernel(x, w, cnt, C_w, C_b.reshape(1, D))
